# baseline (speedup 1.0000x reference)
import functools
import numpy as np
import jax
import jax.numpy as jnp
from jax import lax
from jax.experimental import pallas as pl
from jax.experimental.pallas import tpu as pltpu
from jax.experimental.pallas import tpu_sc as plsc

_B = 128
_DP = 32
_K = 32
_NW = 32
_RPW = _B // _NW
_CPAD = 304
_NCV = _CPAD // 16
_BIGC = 1 << 20


def _tables():
    tris = [(a, b, c)
            for a in range(_DP) for b in range(_DP) for c in range(_DP)
            if (a + 1) * (b + 1) * (c + 1) <= _K]
    t = np.array(tris, np.int32)
    c = t.shape[0]
    t = np.concatenate([t, np.full((_CPAD - c, 3), _DP - 1, np.int32)], 0)
    flat = np.stack([t[:, 0], t[:, 1] + 32, t[:, 2] + 64])
    return flat, c


_TAB, _C = _tables()

_GDN = lax.GatherDimensionNumbers(
    offset_dims=(), collapsed_slice_dims=(0,), start_index_map=(0,))


def _lane_perm(v, idx):
    return lax.gather(v, idx[:, None], _GDN, slice_sizes=(1,),
                      mode=lax.GatherScatterMode.PROMISE_IN_BOUNDS)


def _lexmax(ak, av, bk, bv):
    c = (ak > bk) | ((ak == bk) & (av < bv))
    return (jnp.where(c, ak, bk), jnp.where(c, av, bv),
            jnp.where(c, bk, ak), jnp.where(c, bv, av))


def _merge32(x, y):
    xk0, xk1, xv0, xv1 = x
    yk0, yk1, yv0, yv1 = y
    ryk0, ryv0 = jnp.flip(yk1), jnp.flip(yv1)
    ryk1, ryv1 = jnp.flip(yk0), jnp.flip(yv0)
    z0k, z0v, _, _ = _lexmax(xk0, xv0, ryk0, ryv0)
    z1k, z1v, _, _ = _lexmax(xk1, xv1, ryk1, ryv1)
    uk, uv, vk, vv = _lexmax(z0k, z0v, z1k, z1v)
    s0k, s0v = plsc.sort_key_val(uk, uv, descending=True)
    s1k, s1v = plsc.sort_key_val(vk, vv, descending=True)
    return (s0k, s1k, s0v, s1v)


def kernel(z, log_tau):
    lt16 = jnp.broadcast_to(log_tau, (16,))
    tab = jnp.asarray(_TAB)
    mesh = plsc.VectorSubcoreMesh(core_axis_name="c", subcore_axis_name="s")

    @functools.partial(
        pl.kernel,
        out_type=[jax.ShapeDtypeStruct((_B, _K), jnp.int32),
                  jax.ShapeDtypeStruct((_B, _K), jnp.float32)],
        mesh=mesh,
        compiler_params=pltpu.CompilerParams(needs_layout_passes=False),
        scratch_types=[
            pltpu.VMEM((_RPW, 96), jnp.float32),
            pltpu.VMEM((96,), jnp.float32),
            pltpu.VMEM((96,), jnp.int32),
            pltpu.VMEM((3, _CPAD), jnp.int32),
            pltpu.VMEM((16,), jnp.float32),
            pltpu.VMEM((_RPW, _K), jnp.float32),
            pltpu.VMEM((_RPW, _K), jnp.int32),
            pltpu.VMEM((32,), jnp.int32),
        ],
    )
    def sc(z_hbm, lt_hbm, tab_hbm, idx_hbm, w_hbm,
           zv, sv, av, tabv, ltv, wst, ist, flg):
        wid = lax.axis_index("s") * 2 + lax.axis_index("c")
        base = wid * _RPW
        pltpu.sync_copy(z_hbm.at[pl.ds(base, _RPW)], zv)
        pltpu.sync_copy(lt_hbm, ltv)
        pltpu.sync_copy(tab_hbm, tabv)
        iota = lax.broadcasted_iota(jnp.int32, (16,), 0)
        tau = jnp.exp(ltv[...])

        def fix32(o, k0, k1, v0, v1):
            snA = plsc.load_gather(sv, [iota + (o + 1)])
            snB = plsc.load_gather(sv, [jnp.minimum(iota + (o + 17), o + 31)])
            anA = plsc.load_gather(av, [iota + (o + 1)])
            anB = plsc.load_gather(av, [jnp.minimum(iota + (o + 17), o + 31)])
            fA = ((k0 == snA) & (v0 > anA)).astype(jnp.int32)
            fB = ((k1 == snB) & (v1 > anB) & (iota < 15)).astype(jnp.int32)
            flg[pl.ds(0, 16)] = fA
            flg[pl.ds(16, 16)] = fB
            fpA = jnp.where(iota > 0,
                            plsc.load_gather(flg, [jnp.maximum(iota - 1, 0)]),
                            0)
            fpB = plsc.load_gather(flg, [iota + 15])
            permA = iota + fA - fpA
            permB = iota + 16 + fB - fpB
            a0f = plsc.load_gather(av, [o + permA])
            a1f = plsc.load_gather(av, [o + permB])
            av[pl.ds(o, 16)] = a0f
            av[pl.ds(o + 16, 16)] = a1f
            return a0f, a1f

        for r in range(_RPW):
            for u in range(3):
                o = u * 32
                x0 = zv[r, pl.ds(o, 16)] / tau
                x1 = zv[r, pl.ds(o + 16, 16)] / tau
                m = jnp.maximum(x0, x1)
                for sh in (8, 4, 2, 1):
                    m = jnp.maximum(m, _lane_perm(m, iota ^ sh))
                e0 = jnp.exp(x0 - m)
                e1 = jnp.exp(x1 - m)
                s = e0 + e1
                for sh in (8, 4, 2, 1):
                    s = s + _lane_perm(s, iota ^ sh)
                p0 = e0 / s
                p1 = e1 / s
                k0, v0 = plsc.sort_key_val(p0, iota, descending=True)
                k1, v1 = plsc.sort_key_val(p1, iota + 16, descending=True)
                rbk = jnp.flip(k1)
                rbv = jnp.flip(v1)
                c0 = k0 >= rbk
                lok = jnp.where(c0, k0, rbk)
                lov = jnp.where(c0, v0, rbv)
                hik = jnp.where(c0, rbk, k0)
                hiv = jnp.where(c0, rbv, v0)
                s0k, s0v = plsc.sort_key_val(lok, lov, descending=True)
                s1k, s1v = plsc.sort_key_val(hik, hiv, descending=True)
                sv[pl.ds(o, 16)] = s0k
                sv[pl.ds(o + 16, 16)] = s1k
                av[pl.ds(o, 16)] = s0v
                av[pl.ds(o + 16, 16)] = s1v
                fix32(o, s0k, s1k, s0v, s1v)

            runs16 = []
            for ci in range(_NCV):
                o = ci * 16
                i0 = tabv[0, pl.ds(o, 16)]
                i1 = tabv[1, pl.ds(o, 16)]
                i2 = tabv[2, pl.ds(o, 16)]
                g0 = plsc.load_gather(sv, [i0])
                g1 = plsc.load_gather(sv, [i1])
                g2 = plsc.load_gather(sv, [i2])
                ga0 = plsc.load_gather(av, [i0])
                ga1 = plsc.load_gather(av, [i1])
                ga2 = plsc.load_gather(av, [i2])
                val = (g0 * g1) * g2
                comb = ga0 * (_DP * _DP) + ga1 * _DP + ga2
                if ci == _NCV - 1:
                    pm = iota >= (_C - o)
                    val = jnp.where(pm, -1.0, val)
                    comb = jnp.where(pm, _BIGC, comb)
                runs16.append(plsc.sort_key_val(val, comb, descending=True))
            runs16.append((jnp.full((16,), -1.0, jnp.float32),
                           jnp.full((16,), _BIGC, jnp.int32)))

            runs32 = []
            for i in range(0, len(runs16), 2):
                ak, av_ = runs16[i]
                bk, bv = runs16[i + 1]
                rbk2, rbv2 = jnp.flip(bk), jnp.flip(bv)
                lok, lov, hik, hiv = _lexmax(ak, av_, rbk2, rbv2)
                s0k, s0v = plsc.sort_key_val(lok, lov, descending=True)
                s1k, s1v = plsc.sort_key_val(hik, hiv, descending=True)
                runs32.append((s0k, s1k, s0v, s1v))
            while len(runs32) > 1:
                nxt = [_merge32(runs32[i], runs32[i + 1])
                       for i in range(0, len(runs32) - 1, 2)]
                if len(runs32) % 2:
                    nxt.append(runs32[-1])
                runs32 = nxt
            a0k, a1k, a0v, a1v = runs32[0]

            sv[pl.ds(0, 16)] = a0k
            sv[pl.ds(16, 16)] = a1k
            av[pl.ds(0, 16)] = a0v
            av[pl.ds(16, 16)] = a1v
            a0f, a1f = fix32(0, a0k, a1k, a0v, a1v)
            wst[r, pl.ds(0, 16)] = a0k
            wst[r, pl.ds(16, 16)] = a1k
            ist[r, pl.ds(0, 16)] = a0f
            ist[r, pl.ds(16, 16)] = a1f

        pltpu.sync_copy(wst, w_hbm.at[pl.ds(base, _RPW)])
        pltpu.sync_copy(ist, idx_hbm.at[pl.ds(base, _RPW)])

    indices, weights = sc(z, lt16, tab)
    return (indices, weights)


# speedup vs baseline: 1.1207x; 1.1207x over previous
import numpy as np
import jax
import jax.numpy as jnp
from jax.experimental import pallas as pl
from jax.experimental.pallas import tpu as pltpu

_B = 128
_U = 3
_DP = 32
_K = 32


def _candidate_rank_tables():
    tris = [(a, b, c)
            for a in range(_DP) for b in range(_DP) for c in range(_DP)
            if (a + 1) * (b + 1) * (c + 1) <= _K]
    tris = np.array(tris, dtype=np.int32)
    c = tris.shape[0]
    cpad = ((c + 127) // 128) * 128
    pad = np.full((cpad - c, 3), _DP - 1, np.int32)
    tris = np.concatenate([tris, pad], axis=0)
    return c, cpad, tris


_C, _CPAD, _TRIS = _candidate_rank_tables()


def _body(z_ref, lt_ref, r0_ref, r1_ref, r2_ref, idx_ref, w_ref):
    tau = jnp.exp(lt_ref[0])
    z = z_ref[:, :]
    lane32 = jax.lax.broadcasted_iota(jnp.int32, (_B, _DP), 1)

    svals = []
    sidxs = []
    for u in range(_U):
        x = z[:, u * _DP:(u + 1) * _DP] / tau
        m = jnp.max(x, axis=1, keepdims=True)
        e = jnp.exp(x - m)
        s = jnp.sum(e, axis=1, keepdims=True)
        p = e / s
        su = jnp.zeros((_B, _DP), jnp.float32)
        au = jnp.zeros((_B, _DP), jnp.int32)
        work = p
        for r in range(_DP):
            mv = jnp.max(work, axis=1, keepdims=True)
            mi = jnp.min(jnp.where(work == mv, lane32, _DP * 2),
                         axis=1, keepdims=True)
            su = jnp.where(lane32 == r, mv, su)
            au = jnp.where(lane32 == r, mi, au)
            work = jnp.where(lane32 == mi, -1.0, work)
        svals.append(su)
        sidxs.append(au)

    vs = []
    cs = []
    for u, r_ref in enumerate((r0_ref, r1_ref, r2_ref)):
        ranks = r_ref[:, :]
        vu = jnp.zeros((_B, _CPAD), jnp.float32)
        iu = jnp.zeros((_B, _CPAD), jnp.int32)
        for i in range(_DP):
            msk = ranks == i
            vu = jnp.where(msk, svals[u][:, i:i + 1], vu)
            iu = jnp.where(msk, sidxs[u][:, i:i + 1], iu)
        vs.append(vu)
        cs.append(iu)

    cand_v = (vs[0] * vs[1]) * vs[2]
    comb = cs[0] * (_DP * _DP) + cs[1] * _DP + cs[2]
    clane = jax.lax.broadcasted_iota(jnp.int32, (_B, _CPAD), 1)
    padm = clane >= _C
    cand_v = jnp.where(padm, -1.0, cand_v)
    comb = jnp.where(padm, 1 << 20, comb)

    w_out = jnp.zeros((_B, _K), jnp.float32)
    i_out = jnp.zeros((_B, _K), jnp.int32)
    for t in range(_K):
        mv = jnp.max(cand_v, axis=1, keepdims=True)
        bi = jnp.min(jnp.where(cand_v == mv, comb, 1 << 20),
                     axis=1, keepdims=True)
        w_out = jnp.where(lane32 == t, mv, w_out)
        i_out = jnp.where(lane32 == t, bi, i_out)
        cand_v = jnp.where((cand_v == mv) & (comb == bi), -1.0, cand_v)

    idx_ref[:, :] = i_out
    w_ref[:, :] = w_out


def kernel(z, log_tau):
    r0 = jnp.asarray(_TRIS[:, 0].reshape(1, _CPAD))
    r1 = jnp.asarray(_TRIS[:, 1].reshape(1, _CPAD))
    r2 = jnp.asarray(_TRIS[:, 2].reshape(1, _CPAD))
    indices, weights = pl.pallas_call(
        _body,
        out_shape=[
            jax.ShapeDtypeStruct((_B, _K), jnp.int32),
            jax.ShapeDtypeStruct((_B, _K), jnp.float32),
        ],
        in_specs=[
            pl.BlockSpec(memory_space=pltpu.VMEM),
            pl.BlockSpec(memory_space=pltpu.SMEM),
            pl.BlockSpec(memory_space=pltpu.VMEM),
            pl.BlockSpec(memory_space=pltpu.VMEM),
            pl.BlockSpec(memory_space=pltpu.VMEM),
        ],
        out_specs=[
            pl.BlockSpec(memory_space=pltpu.VMEM),
            pl.BlockSpec(memory_space=pltpu.VMEM),
        ],
    )(z, log_tau, r0, r1, r2)
    return (indices, weights)
